# Initial kernel scaffold; baseline (speedup 1.0000x reference)
import jax
import jax.numpy as jnp
from jax.experimental import pallas as pl


def kernel(visit_concept_orders, pos_encoding):
    raise NotImplementedError("write your pallas kernel here")


# speedup vs baseline: 5.2465x; 5.2465x over previous
import functools

import jax
import jax.numpy as jnp
from jax import lax
from jax.experimental import pallas as pl
from jax.experimental.pallas import tpu as pltpu, tpu_sc as plsc

NC, NS, L = 2, 16, 16
NW = NC * NS


@functools.lru_cache(maxsize=None)
def _make_sc_gather(B, T, V, D):
    assert B % NW == 0 and T % 2 == 0
    rows_w = B // NW
    flat_w = rows_w * T
    CH = 128
    assert flat_w % CH == 0
    nch = flat_w // CH
    assert nch % 2 == 0
    pair_sz = 2 * T
    nslice = pair_sz // L
    half = (T % L)
    assert T // L * L + half == T and 0 < half < L

    mesh = plsc.VectorSubcoreMesh(core_axis_name="c", subcore_axis_name="s")

    @functools.partial(
        pl.kernel,
        out_type=jax.ShapeDtypeStruct((B * T, D), jnp.float32),
        mesh=mesh,
        scratch_types=[
            pltpu.VMEM((flat_w,), jnp.int32),
            pltpu.VMEM((CH, D), jnp.float32),
            pltpu.VMEM((CH, D), jnp.float32),
            pltpu.SemaphoreType.DMA,
            pltpu.SemaphoreType.DMA,
        ],
    )
    def sc_gather(vco_hbm, table_hbm, out_hbm, idx_v, buf_a, buf_b, sem_a, sem_b):
        wid = lax.axis_index("s") * NC + lax.axis_index("c")
        fbase = wid * flat_w
        pltpu.sync_copy(vco_hbm.at[pl.ds(fbase, flat_w)], idx_v)

        lane = lax.iota(jnp.int32, L)
        intmax = jnp.int32(2**31 - 1)
        jmid = T // L

        gdn = lax.GatherDimensionNumbers(
            offset_dims=(), collapsed_slice_dims=(0,), start_index_map=(0,))

        def shuffle(v, idx):
            return lax.gather(
                v, idx[:, None], dimension_numbers=gdn, slice_sizes=(1,),
                mode=lax.GatherScatterMode.PROMISE_IN_BOUNDS)

        def bcast_min(v):
            for s in (8, 4, 2, 1):
                v = jnp.minimum(v, shuffle(v, lane ^ s))
            return v

        def pair_body(p, carry):
            off = p * pair_sz
            vs = [idx_v[pl.ds(off + L * j, L)] for j in range(nslice)]
            mid_a = jnp.where(lane < half, vs[jmid], intmax)
            mid_b = jnp.where(lane >= half, vs[jmid], intmax)
            acc_a = mid_a
            for j in range(jmid):
                acc_a = jnp.minimum(acc_a, vs[j])
            acc_b = mid_b
            for j in range(jmid + 1, nslice):
                acc_b = jnp.minimum(acc_b, vs[j])
            min_a = bcast_min(acc_a)
            min_b = bcast_min(acc_b)
            for j in range(jmid):
                idx_v[pl.ds(off + L * j, L)] = vs[j] - min_a
            mid_m = jnp.where(lane < half, min_a, min_b)
            idx_v[pl.ds(off + L * jmid, L)] = vs[jmid] - mid_m
            for j in range(jmid + 1, nslice):
                idx_v[pl.ds(off + L * j, L)] = vs[j] - min_b
            return carry

        lax.fori_loop(0, rows_w // 2, pair_body, 0)

        def gather(c, buf, sem):
            return pltpu.async_copy(
                table_hbm.at[idx_v.at[pl.ds(c * CH, CH)]], buf, sem)

        def wait(c, buf, sem):
            pltpu.make_async_copy(
                table_hbm.at[idx_v.at[pl.ds(c * CH, CH)]], buf, sem).wait()

        def flush(c, buf):
            pltpu.sync_copy(buf, out_hbm.at[pl.ds(fbase + c * CH, CH)])

        gather(0, buf_a, sem_a)

        def g_body(i, carry):
            c0 = 2 * i
            gather(c0 + 1, buf_b, sem_b)
            wait(c0, buf_a, sem_a)
            flush(c0, buf_a)

            @pl.when(c0 + 2 < nch)
            def _():
                gather(c0 + 2, buf_a, sem_a)

            wait(c0 + 1, buf_b, sem_b)
            flush(c0 + 1, buf_b)
            return carry

        lax.fori_loop(0, nch // 2, g_body, 0)

    return sc_gather


def kernel(visit_concept_orders, pos_encoding):
    B, T = visit_concept_orders.shape
    V, D = pos_encoding.shape
    vco_flat = visit_concept_orders.reshape(B * T)
    fn = _make_sc_gather(B, T, V, D)
    out = fn(vco_flat, pos_encoding)
    return out.reshape(B, T, D)
